# baseline (speedup 1.0000x reference)
import functools

import jax
import jax.numpy as jnp
from jax import lax
from jax.experimental import pallas as pl
from jax.experimental.pallas import tpu as pltpu
from jax.experimental.pallas import tpu_sc as plsc

EPS = 1e-5
G = 256
NC, NS, L = 2, 16, 16
NW = NC * NS
C = 128
HH = 64


def _mesh():
    return plsc.VectorSubcoreMesh(core_axis_name="c", subcore_axis_name="s")


def _make_deg_kernel(N, NPAD, KCH):
    RPT = NPAD // NS

    @functools.partial(
        pl.kernel,
        out_type=(
            jax.ShapeDtypeStruct((NC, NPAD), jnp.float32),
            jax.ShapeDtypeStruct((NC, NPAD), jnp.float32),
        ),
        mesh=_mesh(),
        scratch_types=[
            pltpu.VMEM((KCH, C), jnp.int32),
            pltpu.VMEM((KCH, C), jnp.int32),
            pltpu.VMEM((KCH, C), jnp.float32),
            pltpu.VMEM((NPAD,), jnp.float32),
            pltpu.VMEM((NPAD,), jnp.float32),
            pltpu.VMEM((NS, RPT), jnp.float32),
            pltpu.VMEM_SHARED((NS, NPAD), jnp.float32),
        ],
        compiler_params=pltpu.CompilerParams(needs_layout_passes=False),
    )
    def deg_kernel(row_h, col_h, att_h, degp_h, lwp_h,
                   rowb, colb, attb, degv, lwv, redb, sh):
        cid = lax.axis_index("c")
        tid = lax.axis_index("s")
        wid = tid * NC + cid

        pltpu.sync_copy(row_h.at[wid], rowb)
        pltpu.sync_copy(col_h.at[wid], colb)
        pltpu.sync_copy(att_h.at[wid], attb)

        def init_body(i, _):
            degv[pl.ds(i * L, L)] = jnp.zeros((L,), jnp.float32)
            lwv[pl.ds(i * L, L)] = jnp.full((L,), -1.0, jnp.float32)
            return 0
        lax.fori_loop(0, NPAD // L, init_body, 0)

        def chunk_body(k, _):
            for j in range(C // L):
                r = rowb[k, pl.ds(j * L, L)]
                c = colb[k, pl.ds(j * L, L)]
                a = attb[k, pl.ds(j * L, L)]
                w = jnp.where(r != c, a, 0.0)
                plsc.addupdate_scatter(degv, [r], w)
                plsc.store_scatter(lwv, [r], a, mask=(r == c))
            return 0
        lax.fori_loop(0, KCH, chunk_body, 0)

        base = tid * RPT
        pltpu.sync_copy(degv, sh.at[tid])
        plsc.subcore_barrier()
        pltpu.sync_copy(sh.at[:, pl.ds(base, RPT)], redb)

        def red_sum(v, _):
            acc = jnp.zeros((L,), jnp.float32)
            for rr in range(NS):
                acc = acc + redb[rr, pl.ds(v * L, L)]
            degv[pl.ds(v * L, L)] = acc
            return 0
        lax.fori_loop(0, RPT // L, red_sum, 0)
        pltpu.sync_copy(degv.at[pl.ds(0, RPT)], degp_h.at[cid, pl.ds(base, RPT)])
        plsc.subcore_barrier()

        pltpu.sync_copy(lwv, sh.at[tid])
        plsc.subcore_barrier()
        pltpu.sync_copy(sh.at[:, pl.ds(base, RPT)], redb)

        def red_max(v, _):
            acc = jnp.full((L,), -1.0, jnp.float32)
            for rr in range(NS):
                acc = jnp.maximum(acc, redb[rr, pl.ds(v * L, L)])
            lwv[pl.ds(v * L, L)] = acc
            return 0
        lax.fori_loop(0, RPT // L, red_max, 0)
        pltpu.sync_copy(lwv.at[pl.ds(0, RPT)], lwp_h.at[cid, pl.ds(base, RPT)])

    return deg_kernel


def _make_edge_kernel(N, NPAD, KCH):
    RPT = NPAD // NS

    @functools.partial(
        pl.kernel,
        out_type=(
            jax.ShapeDtypeStruct((NC, NPAD, HH), jnp.float32),
            jax.ShapeDtypeStruct((NC, NPAD, HH), jnp.float32),
        ),
        mesh=_mesh(),
        scratch_types=[
            pltpu.VMEM((KCH, C), jnp.int32),
            pltpu.VMEM((KCH, C), jnp.int32),
            pltpu.VMEM((KCH, C), jnp.float32),
            pltpu.VMEM((C, HH), jnp.float32),
            pltpu.VMEM((C, HH), jnp.float32),
            pltpu.VMEM_SHARED((NPAD, HH), jnp.float32),
            pltpu.SemaphoreType.DMA,
            pltpu.SemaphoreType.DMA,
        ],
        compiler_params=pltpu.CompilerParams(
            needs_layout_passes=False, use_tc_tiling_on_sc=False),
    )
    def edge_kernel(row_h, col_h, att_h, g0_h, g1_h, accA_h, accB_h,
                    rowb, colb, wb, gb0, gb1, acc_sh, gs0, gs1):
        cid = lax.axis_index("c")
        tid = lax.axis_index("s")
        wid = tid * NC + cid
        base = tid * RPT

        pltpu.sync_copy(row_h.at[wid], rowb)
        pltpu.sync_copy(col_h.at[wid], colb)
        pltpu.sync_copy(att_h.at[wid], wb)

        def w_body(k, _):
            for j in range(C // L):
                r = rowb[k, pl.ds(j * L, L)]
                c = colb[k, pl.ds(j * L, L)]
                a = wb[k, pl.ds(j * L, L)]
                wb[k, pl.ds(j * L, L)] = jnp.where(r != c, a, 0.0)
            return 0
        lax.fori_loop(0, KCH, w_body, 0)

        gbs = (gb0, gb1)
        gsems = (gs0, gs1)

        def one_pass(g_h, acc_h):
            def z_body(i, _):
                for j in range(HH // L):
                    gb0[i, pl.ds(j * L, L)] = jnp.zeros((L,), jnp.float32)
                return 0
            lax.fori_loop(0, C, z_body, 0)
            for mb in range(RPT // C):
                pltpu.sync_copy(gb0, acc_sh.at[pl.ds(base + mb * C, C)])
            plsc.subcore_barrier()

            pltpu.async_copy(g_h.at[rowb.at[0]], gb0, gs0)
            pltpu.async_copy(g_h.at[rowb.at[1]], gb1, gs1)

            def do_chunk(k, bb):
                gbuf, gsem = gbs[bb], gsems[bb]
                pltpu.make_async_copy(g_h.at[rowb.at[k]], gbuf, gsem).wait()

                def e_body(e, _):
                    ws = plsc.load_gather(
                        wb, [jnp.full((L,), k, jnp.int32),
                             jnp.full((L,), e, jnp.int32)])
                    for j in range(HH // L):
                        gbuf[e, pl.ds(j * L, L)] = gbuf[e, pl.ds(j * L, L)] * ws
                    return 0
                lax.fori_loop(0, C, e_body, 0)

                pltpu.sync_copy(gbuf, acc_sh.at[colb.at[k]], add=True)

                @pl.when(k + 2 < KCH)
                def _():
                    pltpu.async_copy(g_h.at[rowb.at[k + 2]], gbuf, gsem)

            def loop2(k2, _):
                for bb in range(2):
                    do_chunk(k2 * 2 + bb, bb)
                return 0
            lax.fori_loop(0, KCH // 2, loop2, 0)
            if KCH % 2 == 1:
                do_chunk(KCH - 1, 0)

            plsc.subcore_barrier()
            pltpu.sync_copy(acc_sh.at[pl.ds(base, RPT)],
                            acc_h.at[cid, pl.ds(base, RPT)])
            plsc.subcore_barrier()

        one_pass(g0_h, accA_h)
        one_pass(g1_h, accB_h)

    return edge_kernel


def _stats_body(x_ref, na_ref, o_ref):
    i = pl.program_id(0)
    xa = na_ref[...] * x_ref[...]

    @pl.when(i == 0)
    def _():
        o_ref[...] = jnp.zeros_like(o_ref)
    o_ref[0:1, :] += jnp.sum(xa, axis=0, keepdims=True)
    o_ref[1:2, :] += jnp.sum(xa * xa, axis=0, keepdims=True)


def _deg_terms(d0, d1, l0, l1):
    lwp = jnp.maximum(l0, l1)
    lw = jnp.where(lwp >= 0.0, lwp, 1.0)
    deg = d0 + d1 + lw
    dinv = jnp.where(deg > 0.0, lax.rsqrt(jnp.maximum(deg, EPS * EPS)), 0.0)
    return lw, dinv


def _make_transform_body(N):
    def body(x_ref, na_ref, st_ref, w_ref, bnw_ref, bnb_ref,
             d0_ref, d1_ref, l0_ref, l1_ref, g0_ref, g1_ref):
        m = st_ref[0:1, :] * (1.0 / N)
        msq = st_ref[1:2, :] * (1.0 / N)
        var = msq - m * m
        s = bnw_ref[...] * lax.rsqrt(var + EPS)
        t = bnb_ref[...] - m * s
        xb = (na_ref[...] * x_ref[...]) * s + t
        h = jnp.dot(xb, w_ref[...], preferred_element_type=jnp.float32)
        _, dinv = _deg_terms(d0_ref[...], d1_ref[...], l0_ref[...], l1_ref[...])
        g = dinv * h
        g0_ref[...] = g[:, :HH]
        g1_ref[...] = g[:, HH:]
    return body


def _pool_body(accA_ref, accB_ref, g0_ref, g1_ref,
               d0_ref, d1_ref, l0_ref, l1_ref, b_ref, bat_ref, o_ref):
    i = pl.program_id(0)
    lw, dinv = _deg_terms(d0_ref[...], d1_ref[...], l0_ref[...], l1_ref[...])
    acc = jnp.concatenate(
        [accA_ref[0] + accA_ref[1], accB_ref[0] + accB_ref[1]], axis=1)
    g = jnp.concatenate([g0_ref[...], g1_ref[...]], axis=1)
    out = dinv * (acc + lw * g) + b_ref[...]
    out = jnp.maximum(out, 0.0)
    oh = (bat_ref[...] == lax.broadcasted_iota(jnp.int32, (1, G), 1))
    p = lax.dot_general(oh.astype(jnp.float32), out,
                        (((0,), (0,)), ((), ())),
                        preferred_element_type=jnp.float32)

    @pl.when(i == 0)
    def _():
        o_ref[...] = jnp.zeros_like(o_ref)
    o_ref[...] += p


def kernel(x, edge_index, batch, edge_att, node_att, W, b, bn_weight, bn_bias):
    N, H = x.shape
    E = edge_att.shape[0]
    KCH = -(-E // (NW * C))
    EP = NW * KCH * C
    NPAD = -(-N // (NS * 128)) * (NS * 128)

    row = edge_index[0]
    col = edge_index[1]
    pad = EP - E
    zpad = jnp.zeros((pad,), jnp.int32)
    row_p = jnp.concatenate([row, zpad]).reshape(NW, KCH, C)
    col_p = jnp.concatenate([col, zpad]).reshape(NW, KCH, C)
    att_p = jnp.concatenate(
        [edge_att, jnp.full((pad,), -1.0, jnp.float32)]).reshape(NW, KCH, C)

    degp, lwp = _make_deg_kernel(N, NPAD, KCH)(row_p, col_p, att_p)

    bN = N // 5
    stats = pl.pallas_call(
        _stats_body,
        grid=(5,),
        in_specs=[pl.BlockSpec((bN, H), lambda i: (i, 0)),
                  pl.BlockSpec((bN, 1), lambda i: (i, 0))],
        out_specs=pl.BlockSpec((8, H), lambda i: (0, 0)),
        out_shape=jax.ShapeDtypeStruct((8, H), jnp.float32),
    )(x, node_att)

    d0 = degp[0, :N].reshape(N, 1)
    d1 = degp[1, :N].reshape(N, 1)
    l0 = lwp[0, :N].reshape(N, 1)
    l1 = lwp[1, :N].reshape(N, 1)

    nspec = pl.BlockSpec((bN, 1), lambda i: (i, 0))
    full2 = lambda shape: pl.BlockSpec(shape, lambda i: (0, 0))
    hspec = pl.BlockSpec((bN, HH), lambda i: (i, 0))
    g0, g1 = pl.pallas_call(
        _make_transform_body(N),
        grid=(5,),
        in_specs=[pl.BlockSpec((bN, H), lambda i: (i, 0)),
                  nspec,
                  full2((8, H)),
                  full2((H, H)),
                  full2((1, H)),
                  full2((1, H)),
                  nspec, nspec, nspec, nspec],
        out_specs=[hspec, hspec],
        out_shape=[jax.ShapeDtypeStruct((N, HH), jnp.float32),
                   jax.ShapeDtypeStruct((N, HH), jnp.float32)],
    )(x, node_att, stats, W, bn_weight.reshape(1, H), bn_bias.reshape(1, H),
      d0, d1, l0, l1)

    accA, accB = _make_edge_kernel(N, NPAD, KCH)(row_p, col_p, att_p, g0, g1)

    aspec = pl.BlockSpec((NC, bN, HH), lambda i: (0, i, 0))
    pooled = pl.pallas_call(
        _pool_body,
        grid=(5,),
        in_specs=[aspec, aspec, hspec, hspec,
                  nspec, nspec, nspec, nspec,
                  full2((1, H)),
                  pl.BlockSpec((bN, 1), lambda i: (i, 0))],
        out_specs=pl.BlockSpec((G, H), lambda i: (0, 0)),
        out_shape=jax.ShapeDtypeStruct((G, H), jnp.float32),
    )(accA, accB, g0, g1, d0, d1, l0, l1,
      b.reshape(1, H), batch.reshape(N, 1))
    return pooled


# speedup vs baseline: 1.1985x; 1.1985x over previous
import functools

import jax
import jax.numpy as jnp
from jax import lax
from jax.experimental import pallas as pl
from jax.experimental.pallas import tpu as pltpu
from jax.experimental.pallas import tpu_sc as plsc

EPS = 1e-5
G = 256
NC, NS, L = 2, 16, 16
NW = NC * NS
C = 128
HH = 64


def _mesh():
    return plsc.VectorSubcoreMesh(core_axis_name="c", subcore_axis_name="s")


def _make_deg_kernel(N, NPAD, KCH):
    RPT = NPAD // NS

    @functools.partial(
        pl.kernel,
        out_type=(
            jax.ShapeDtypeStruct((NC, NPAD), jnp.float32),
            jax.ShapeDtypeStruct((NC, NPAD), jnp.float32),
        ),
        mesh=_mesh(),
        scratch_types=[
            pltpu.VMEM((KCH, C), jnp.int32),
            pltpu.VMEM((KCH, C), jnp.int32),
            pltpu.VMEM((KCH, C), jnp.float32),
            pltpu.VMEM((NPAD,), jnp.float32),
            pltpu.VMEM((NPAD,), jnp.float32),
            pltpu.VMEM((NS, RPT), jnp.float32),
            pltpu.VMEM_SHARED((NS, NPAD), jnp.float32),
        ],
        compiler_params=pltpu.CompilerParams(needs_layout_passes=False),
    )
    def deg_kernel(row_h, col_h, att_h, degp_h, lwp_h,
                   rowb, colb, attb, degv, lwv, redb, sh):
        cid = lax.axis_index("c")
        tid = lax.axis_index("s")
        wid = tid * NC + cid

        pltpu.sync_copy(row_h.at[wid], rowb)
        pltpu.sync_copy(col_h.at[wid], colb)
        pltpu.sync_copy(att_h.at[wid], attb)

        def init_body(i, _):
            degv[pl.ds(i * L, L)] = jnp.zeros((L,), jnp.float32)
            lwv[pl.ds(i * L, L)] = jnp.full((L,), -1.0, jnp.float32)
            return 0
        lax.fori_loop(0, NPAD // L, init_body, 0)

        def chunk_body(k, _):
            for j in range(C // L):
                r = rowb[k, pl.ds(j * L, L)]
                c = colb[k, pl.ds(j * L, L)]
                a = attb[k, pl.ds(j * L, L)]
                w = jnp.where(r != c, a, 0.0)
                plsc.addupdate_scatter(degv, [r], w)
                plsc.store_scatter(lwv, [r], a, mask=(r == c))
            return 0
        lax.fori_loop(0, KCH, chunk_body, 0)

        base = tid * RPT
        pltpu.sync_copy(degv, sh.at[tid])
        plsc.subcore_barrier()
        pltpu.sync_copy(sh.at[:, pl.ds(base, RPT)], redb)

        def red_sum(v, _):
            acc = jnp.zeros((L,), jnp.float32)
            for rr in range(NS):
                acc = acc + redb[rr, pl.ds(v * L, L)]
            degv[pl.ds(v * L, L)] = acc
            return 0
        lax.fori_loop(0, RPT // L, red_sum, 0)
        pltpu.sync_copy(degv.at[pl.ds(0, RPT)], degp_h.at[cid, pl.ds(base, RPT)])
        plsc.subcore_barrier()

        pltpu.sync_copy(lwv, sh.at[tid])
        plsc.subcore_barrier()
        pltpu.sync_copy(sh.at[:, pl.ds(base, RPT)], redb)

        def red_max(v, _):
            acc = jnp.full((L,), -1.0, jnp.float32)
            for rr in range(NS):
                acc = jnp.maximum(acc, redb[rr, pl.ds(v * L, L)])
            lwv[pl.ds(v * L, L)] = acc
            return 0
        lax.fori_loop(0, RPT // L, red_max, 0)
        pltpu.sync_copy(lwv.at[pl.ds(0, RPT)], lwp_h.at[cid, pl.ds(base, RPT)])

    return deg_kernel


C2 = 80


def _make_edge_kernel(N, NPAD, KCH2, H):
    RPT = NPAD // NS

    @functools.partial(
        pl.kernel,
        out_type=jax.ShapeDtypeStruct((NC, NPAD, H), jnp.float32),
        mesh=_mesh(),
        scratch_types=[
            pltpu.VMEM((KCH2, C2), jnp.int32),
            pltpu.VMEM((KCH2, C2), jnp.float32),
            pltpu.VMEM((2, C2), jnp.int32),
            pltpu.VMEM((2, C2), jnp.int32),
            pltpu.VMEM((2, C2), jnp.float32),
            pltpu.VMEM((C2, H), jnp.float32),
            pltpu.VMEM((C2, H), jnp.float32),
            pltpu.VMEM_SHARED((NPAD, H), jnp.float32),
            pltpu.SemaphoreType.DMA,
            pltpu.SemaphoreType.DMA,
        ],
        compiler_params=pltpu.CompilerParams(
            needs_layout_passes=False, use_tc_tiling_on_sc=False),
    )
    def edge_kernel(pk_h, att_h, g_h, acc_h,
                    pki, attb, rowi, coli, wbuf, gb0, gb1, acc_sh, gs0, gs1):
        cid = lax.axis_index("c")
        tid = lax.axis_index("s")
        wid = tid * NC + cid
        base = tid * RPT

        pltpu.sync_copy(pk_h.at[wid], pki)
        pltpu.sync_copy(att_h.at[wid], attb)

        def unpack(k, bb):
            for j in range(C2 // L):
                p = pki[k, pl.ds(j * L, L)]
                r = lax.bitwise_and(p, 16383)
                c = lax.shift_right_logical(p, 14)
                a = attb[k, pl.ds(j * L, L)]
                rowi[bb, pl.ds(j * L, L)] = r
                coli[bb, pl.ds(j * L, L)] = c
                wbuf[bb, pl.ds(j * L, L)] = jnp.where(r != c, a, 0.0)

        def z_body(i, _):
            for j in range(H // L):
                gb0[i, pl.ds(j * L, L)] = jnp.zeros((L,), jnp.float32)
            return 0
        lax.fori_loop(0, C2, z_body, 0)

        def zc_body(mb, _):
            off = pl.multiple_of(base + mb * C2, C2)
            pltpu.sync_copy(gb0, acc_sh.at[pl.ds(off, C2)])
            return 0
        lax.fori_loop(0, RPT // C2, zc_body, 0)
        plsc.subcore_barrier()

        gbs = (gb0, gb1)
        gsems = (gs0, gs1)

        unpack(0, 0)
        pltpu.async_copy(g_h.at[rowi.at[0]], gb0, gs0)
        unpack(1, 1)
        pltpu.async_copy(g_h.at[rowi.at[1]], gb1, gs1)

        def do_chunk(k, bb):
            gbuf, gsem = gbs[bb], gsems[bb]
            pltpu.make_async_copy(g_h.at[rowi.at[bb]], gbuf, gsem).wait()

            def e_body(e, _):
                ws = plsc.load_gather(
                    wbuf, [jnp.full((L,), bb, jnp.int32),
                           jnp.full((L,), e, jnp.int32)])
                for j in range(H // L):
                    gbuf[e, pl.ds(j * L, L)] = gbuf[e, pl.ds(j * L, L)] * ws
                return 0
            lax.fori_loop(0, C2, e_body, 0)

            pltpu.sync_copy(gbuf, acc_sh.at[coli.at[bb]], add=True)

            @pl.when(k + 2 < KCH2)
            def _():
                unpack(k + 2, bb)
                pltpu.async_copy(g_h.at[rowi.at[bb]], gbuf, gsem)

        def loop2(k2, _):
            for bb in range(2):
                do_chunk(k2 * 2 + bb, bb)
            return 0
        lax.fori_loop(0, KCH2 // 2, loop2, 0)

        plsc.subcore_barrier()

        def rb_body(nb, _):
            off = pl.multiple_of(base + nb * 128, 128)
            pltpu.sync_copy(acc_sh.at[pl.ds(off, 128)],
                            acc_h.at[cid, pl.ds(off, 128)])
            return 0
        lax.fori_loop(0, RPT // 128, rb_body, 0)

    return edge_kernel


def _stats_body(x_ref, na_ref, o_ref):
    i = pl.program_id(0)
    xa = na_ref[...] * x_ref[...]

    @pl.when(i == 0)
    def _():
        o_ref[...] = jnp.zeros_like(o_ref)
    o_ref[0:1, :] += jnp.sum(xa, axis=0, keepdims=True)
    o_ref[1:2, :] += jnp.sum(xa * xa, axis=0, keepdims=True)


def _deg_terms(d0, d1, l0, l1):
    lwp = jnp.maximum(l0, l1)
    lw = jnp.where(lwp >= 0.0, lwp, 1.0)
    deg = d0 + d1 + lw
    dinv = jnp.where(deg > 0.0, lax.rsqrt(jnp.maximum(deg, EPS * EPS)), 0.0)
    return lw, dinv


def _make_transform_body(N):
    def body(x_ref, na_ref, st_ref, w_ref, bnw_ref, bnb_ref,
             d0_ref, d1_ref, l0_ref, l1_ref, g_ref):
        m = st_ref[0:1, :] * (1.0 / N)
        msq = st_ref[1:2, :] * (1.0 / N)
        var = msq - m * m
        s = bnw_ref[...] * lax.rsqrt(var + EPS)
        t = bnb_ref[...] - m * s
        xb = (na_ref[...] * x_ref[...]) * s + t
        h = jnp.dot(xb, w_ref[...], preferred_element_type=jnp.float32)
        _, dinv = _deg_terms(d0_ref[...], d1_ref[...], l0_ref[...], l1_ref[...])
        g_ref[...] = dinv * h
    return body


def _pool_body(acc_ref, g_ref,
               d0_ref, d1_ref, l0_ref, l1_ref, b_ref, bat_ref, o_ref):
    i = pl.program_id(0)
    lw, dinv = _deg_terms(d0_ref[...], d1_ref[...], l0_ref[...], l1_ref[...])
    acc = acc_ref[0] + acc_ref[1]
    out = dinv * (acc + lw * g_ref[...]) + b_ref[...]
    out = jnp.maximum(out, 0.0)
    oh = (bat_ref[...] == lax.broadcasted_iota(jnp.int32, (1, G), 1))
    p = lax.dot_general(oh.astype(jnp.float32), out,
                        (((0,), (0,)), ((), ())),
                        preferred_element_type=jnp.float32)

    @pl.when(i == 0)
    def _():
        o_ref[...] = jnp.zeros_like(o_ref)
    o_ref[...] += p


def kernel(x, edge_index, batch, edge_att, node_att, W, b, bn_weight, bn_bias):
    N, H = x.shape
    E = edge_att.shape[0]
    KCH = -(-E // (NW * C))
    EP = NW * KCH * C
    NPAD = -(-N // (NS * 128)) * (NS * 128)

    row = edge_index[0]
    col = edge_index[1]
    pad = EP - E
    zpad = jnp.zeros((pad,), jnp.int32)
    row_p = jnp.concatenate([row, zpad]).reshape(NW, KCH, C)
    col_p = jnp.concatenate([col, zpad]).reshape(NW, KCH, C)
    att_p = jnp.concatenate(
        [edge_att, jnp.full((pad,), -1.0, jnp.float32)]).reshape(NW, KCH, C)

    degp, lwp = _make_deg_kernel(N, NPAD, KCH)(row_p, col_p, att_p)

    bN = N // 5
    stats = pl.pallas_call(
        _stats_body,
        grid=(5,),
        in_specs=[pl.BlockSpec((bN, H), lambda i: (i, 0)),
                  pl.BlockSpec((bN, 1), lambda i: (i, 0))],
        out_specs=pl.BlockSpec((8, H), lambda i: (0, 0)),
        out_shape=jax.ShapeDtypeStruct((8, H), jnp.float32),
    )(x, node_att)

    d0 = degp[0, :N].reshape(N, 1)
    d1 = degp[1, :N].reshape(N, 1)
    l0 = lwp[0, :N].reshape(N, 1)
    l1 = lwp[1, :N].reshape(N, 1)

    nspec = pl.BlockSpec((bN, 1), lambda i: (i, 0))
    full2 = lambda shape: pl.BlockSpec(shape, lambda i: (0, 0))
    g = pl.pallas_call(
        _make_transform_body(N),
        grid=(5,),
        in_specs=[pl.BlockSpec((bN, H), lambda i: (i, 0)),
                  nspec,
                  full2((8, H)),
                  full2((H, H)),
                  full2((1, H)),
                  full2((1, H)),
                  nspec, nspec, nspec, nspec],
        out_specs=pl.BlockSpec((bN, H), lambda i: (i, 0)),
        out_shape=jax.ShapeDtypeStruct((N, H), jnp.float32),
    )(x, node_att, stats, W, bn_weight.reshape(1, H), bn_bias.reshape(1, H),
      d0, d1, l0, l1)

    KCH2 = -(-E // (NW * C2))
    KCH2 += KCH2 % 2
    EP2 = NW * KCH2 * C2
    zpad2 = jnp.zeros((EP2 - E,), jnp.int32)
    pk_p = jnp.concatenate(
        [row | (col << 14), zpad2]).reshape(NW, KCH2, C2)
    att2_p = jnp.concatenate(
        [edge_att, jnp.full((EP2 - E,), -1.0, jnp.float32)]
    ).reshape(NW, KCH2, C2)

    acc = _make_edge_kernel(N, NPAD, KCH2, H)(pk_p, att2_p, g)

    aspec = pl.BlockSpec((NC, bN, H), lambda i: (0, i, 0))
    pooled = pl.pallas_call(
        _pool_body,
        grid=(5,),
        in_specs=[aspec,
                  pl.BlockSpec((bN, H), lambda i: (i, 0)),
                  nspec, nspec, nspec, nspec,
                  full2((1, H)),
                  pl.BlockSpec((bN, 1), lambda i: (i, 0))],
        out_specs=pl.BlockSpec((G, H), lambda i: (0, 0)),
        out_shape=jax.ShapeDtypeStruct((G, H), jnp.float32),
    )(acc, g, d0, d1, l0, l1,
      b.reshape(1, H), batch.reshape(N, 1))
    return pooled


# speedup vs baseline: 1.3258x; 1.1062x over previous
import functools

import jax
import jax.numpy as jnp
from jax import lax
from jax.experimental import pallas as pl
from jax.experimental.pallas import tpu as pltpu
from jax.experimental.pallas import tpu_sc as plsc

EPS = 1e-5
G = 256
NC, NS, L = 2, 16, 16
NW = NC * NS
C = 128
HH = 64


def _mesh():
    return plsc.VectorSubcoreMesh(core_axis_name="c", subcore_axis_name="s")


def _make_deg_kernel(N, NPAD, KCH):
    RPT = NPAD // NS

    @functools.partial(
        pl.kernel,
        out_type=(
            jax.ShapeDtypeStruct((NC, NPAD), jnp.float32),
            jax.ShapeDtypeStruct((NC, NPAD), jnp.float32),
        ),
        mesh=_mesh(),
        scratch_types=[
            pltpu.VMEM((KCH, C), jnp.int32),
            pltpu.VMEM((KCH, C), jnp.int32),
            pltpu.VMEM((KCH, C), jnp.float32),
            pltpu.VMEM((NPAD,), jnp.float32),
            pltpu.VMEM((NPAD,), jnp.float32),
            pltpu.VMEM((NS, RPT), jnp.float32),
            pltpu.VMEM_SHARED((NS, NPAD), jnp.float32),
        ],
        compiler_params=pltpu.CompilerParams(needs_layout_passes=False),
    )
    def deg_kernel(row_h, col_h, att_h, degp_h, lwp_h,
                   rowb, colb, attb, degv, lwv, redb, sh):
        cid = lax.axis_index("c")
        tid = lax.axis_index("s")
        wid = tid * NC + cid

        pltpu.sync_copy(row_h.at[wid], rowb)
        pltpu.sync_copy(col_h.at[wid], colb)
        pltpu.sync_copy(att_h.at[wid], attb)

        def init_body(i, _):
            degv[pl.ds(i * L, L)] = jnp.zeros((L,), jnp.float32)
            lwv[pl.ds(i * L, L)] = jnp.full((L,), -1.0, jnp.float32)
            return 0
        lax.fori_loop(0, NPAD // L, init_body, 0)

        def chunk_body(k, _):
            for j in range(C // L):
                r = rowb[k, pl.ds(j * L, L)]
                c = colb[k, pl.ds(j * L, L)]
                a = attb[k, pl.ds(j * L, L)]
                w = jnp.where(r != c, a, 0.0)
                plsc.addupdate_scatter(degv, [r], w)
                plsc.store_scatter(lwv, [r], a, mask=(r == c))
            return 0
        lax.fori_loop(0, KCH, chunk_body, 0)

        base = tid * RPT
        pltpu.sync_copy(degv, sh.at[tid])
        plsc.subcore_barrier()
        pltpu.sync_copy(sh.at[:, pl.ds(base, RPT)], redb)

        def red_sum(v, _):
            acc = jnp.zeros((L,), jnp.float32)
            for rr in range(NS):
                acc = acc + redb[rr, pl.ds(v * L, L)]
            degv[pl.ds(v * L, L)] = acc
            return 0
        lax.fori_loop(0, RPT // L, red_sum, 0)
        pltpu.sync_copy(degv.at[pl.ds(0, RPT)], degp_h.at[cid, pl.ds(base, RPT)])
        plsc.subcore_barrier()

        pltpu.sync_copy(lwv, sh.at[tid])
        plsc.subcore_barrier()
        pltpu.sync_copy(sh.at[:, pl.ds(base, RPT)], redb)

        def red_max(v, _):
            acc = jnp.full((L,), -1.0, jnp.float32)
            for rr in range(NS):
                acc = jnp.maximum(acc, redb[rr, pl.ds(v * L, L)])
            lwv[pl.ds(v * L, L)] = acc
            return 0
        lax.fori_loop(0, RPT // L, red_max, 0)
        pltpu.sync_copy(lwv.at[pl.ds(0, RPT)], lwp_h.at[cid, pl.ds(base, RPT)])

    return deg_kernel


C2 = 80


def _make_edge_kernel(N, NPAD, KA, KB, H):
    RPT = NPAD // NS
    KMAX = max(KA, KB)

    @functools.partial(
        pl.kernel,
        out_type=jax.ShapeDtypeStruct((NC, NPAD, H), jnp.float32),
        mesh=_mesh(),
        scratch_types=[
            pltpu.VMEM((KMAX, C2), jnp.int32),
            pltpu.VMEM((KMAX, C2), jnp.float32),
            pltpu.VMEM((2, C2), jnp.int32),
            pltpu.VMEM((2, C2), jnp.int32),
            pltpu.VMEM((2, C2), jnp.float32),
            pltpu.VMEM((C2, H), jnp.float32),
            pltpu.VMEM((C2, H), jnp.float32),
            pltpu.VMEM_SHARED((NPAD, H), jnp.float32),
            pltpu.SemaphoreType.DMA,
            pltpu.SemaphoreType.DMA,
        ],
        compiler_params=pltpu.CompilerParams(
            needs_layout_passes=False, use_tc_tiling_on_sc=False),
    )
    def edge_kernel(pk_h, att_h, g_h, acc_h,
                    pki, attb, rowi, coli, wbuf, gb0, gb1, acc_sh, gs0, gs1):
        cid = lax.axis_index("c")
        tid = lax.axis_index("s")
        wid = tid * NC + cid
        base = tid * RPT
        kw = jnp.where(cid == 0, KA, KB)

        pltpu.sync_copy(pk_h.at[wid], pki)
        pltpu.sync_copy(att_h.at[wid], attb)

        def unpack(k, bb):
            for j in range(C2 // L):
                p = pki[k, pl.ds(j * L, L)]
                r = lax.bitwise_and(p, 16383)
                c = lax.shift_right_logical(p, 14)
                a = attb[k, pl.ds(j * L, L)]
                rowi[bb, pl.ds(j * L, L)] = r
                coli[bb, pl.ds(j * L, L)] = c
                wbuf[bb, pl.ds(j * L, L)] = jnp.where(r != c, a, 0.0)

        def z_body(i, _):
            for j in range(H // L):
                gb0[i, pl.ds(j * L, L)] = jnp.zeros((L,), jnp.float32)
            return 0
        lax.fori_loop(0, C2, z_body, 0)

        def zc_body(mb, _):
            off = pl.multiple_of(base + mb * C2, C2)
            pltpu.sync_copy(gb0, acc_sh.at[pl.ds(off, C2)])
            return 0
        lax.fori_loop(0, RPT // C2, zc_body, 0)
        plsc.subcore_barrier()

        gbs = (gb0, gb1)
        gsems = (gs0, gs1)

        unpack(0, 0)
        pltpu.async_copy(g_h.at[rowi.at[0]], gb0, gs0)
        unpack(1, 1)
        pltpu.async_copy(g_h.at[rowi.at[1]], gb1, gs1)

        def do_chunk(k, bb):
            gbuf, gsem = gbs[bb], gsems[bb]
            pltpu.make_async_copy(g_h.at[rowi.at[bb]], gbuf, gsem).wait()

            def e_body(e, _):
                ws = plsc.load_gather(
                    wbuf, [jnp.full((L,), bb, jnp.int32),
                           jnp.full((L,), e, jnp.int32)])
                for j in range(H // L):
                    gbuf[e, pl.ds(j * L, L)] = gbuf[e, pl.ds(j * L, L)] * ws
                return 0
            lax.fori_loop(0, C2, e_body, 0)

            pltpu.sync_copy(gbuf, acc_sh.at[coli.at[bb]], add=True)

            @pl.when(k + 2 < kw)
            def _():
                unpack(k + 2, bb)
                pltpu.async_copy(g_h.at[rowi.at[bb]], gbuf, gsem)

        def loop2(k2, _):
            for bb in range(2):
                do_chunk(k2 * 2 + bb, bb)
            return 0
        lax.fori_loop(0, kw // 2, loop2, 0)

        plsc.subcore_barrier()

        def rb_body(nb, _):
            off = pl.multiple_of(base + nb * 128, 128)
            pltpu.sync_copy(acc_sh.at[pl.ds(off, 128)],
                            acc_h.at[cid, pl.ds(off, 128)])
            return 0
        lax.fori_loop(0, RPT // 128, rb_body, 0)

    return edge_kernel


def _stats_body(x_ref, na_ref, o_ref):
    i = pl.program_id(0)
    xa = na_ref[...] * x_ref[...]

    @pl.when(i == 0)
    def _():
        o_ref[...] = jnp.zeros_like(o_ref)
    o_ref[0:1, :] += jnp.sum(xa, axis=0, keepdims=True)
    o_ref[1:2, :] += jnp.sum(xa * xa, axis=0, keepdims=True)


def _deg_terms(d0, d1, l0, l1):
    lwp = jnp.maximum(l0, l1)
    lw = jnp.where(lwp >= 0.0, lwp, 1.0)
    deg = d0 + d1 + lw
    dinv = jnp.where(deg > 0.0, lax.rsqrt(jnp.maximum(deg, EPS * EPS)), 0.0)
    return lw, dinv


def _make_transform_body(N):
    def body(x_ref, na_ref, st_ref, w_ref, bnw_ref, bnb_ref,
             d0_ref, d1_ref, l0_ref, l1_ref, g_ref):
        m = st_ref[0:1, :] * (1.0 / N)
        msq = st_ref[1:2, :] * (1.0 / N)
        var = msq - m * m
        s = bnw_ref[...] * lax.rsqrt(var + EPS)
        t = bnb_ref[...] - m * s
        xb = (na_ref[...] * x_ref[...]) * s + t
        h = jnp.dot(xb, w_ref[...], preferred_element_type=jnp.float32)
        _, dinv = _deg_terms(d0_ref[...], d1_ref[...], l0_ref[...], l1_ref[...])
        g_ref[...] = dinv * h
    return body


def _pool_body(acc_ref, g_ref,
               d0_ref, d1_ref, l0_ref, l1_ref, b_ref, bat_ref, o_ref):
    i = pl.program_id(0)
    lw, dinv = _deg_terms(d0_ref[...], d1_ref[...], l0_ref[...], l1_ref[...])
    acc = acc_ref[0] + acc_ref[1]
    out = dinv * (acc + lw * g_ref[...]) + b_ref[...]
    out = jnp.maximum(out, 0.0)
    oh = (bat_ref[...] == lax.broadcasted_iota(jnp.int32, (1, G), 1))
    p = lax.dot_general(oh.astype(jnp.float32), out,
                        (((0,), (0,)), ((), ())),
                        preferred_element_type=jnp.float32)

    @pl.when(i == 0)
    def _():
        o_ref[...] = jnp.zeros_like(o_ref)
    o_ref[...] += p


def kernel(x, edge_index, batch, edge_att, node_att, W, b, bn_weight, bn_bias):
    N, H = x.shape
    E = edge_att.shape[0]
    KCH = -(-E // (NW * C))
    EP = NW * KCH * C
    NPAD = -(-N // (NS * 128)) * (NS * 128)

    row = edge_index[0]
    col = edge_index[1]
    pad = EP - E
    zpad = jnp.zeros((pad,), jnp.int32)
    row_p = jnp.concatenate([row, zpad]).reshape(NW, KCH, C)
    col_p = jnp.concatenate([col, zpad]).reshape(NW, KCH, C)
    att_p = jnp.concatenate(
        [edge_att, jnp.full((pad,), -1.0, jnp.float32)]).reshape(NW, KCH, C)

    degp, lwp = _make_deg_kernel(N, NPAD, KCH)(row_p, col_p, att_p)

    bN = N // 5
    stats = pl.pallas_call(
        _stats_body,
        grid=(5,),
        in_specs=[pl.BlockSpec((bN, H), lambda i: (i, 0)),
                  pl.BlockSpec((bN, 1), lambda i: (i, 0))],
        out_specs=pl.BlockSpec((8, H), lambda i: (0, 0)),
        out_shape=jax.ShapeDtypeStruct((8, H), jnp.float32),
    )(x, node_att)

    d0 = degp[0, :N].reshape(N, 1)
    d1 = degp[1, :N].reshape(N, 1)
    l0 = lwp[0, :N].reshape(N, 1)
    l1 = lwp[1, :N].reshape(N, 1)

    nspec = pl.BlockSpec((bN, 1), lambda i: (i, 0))
    full2 = lambda shape: pl.BlockSpec(shape, lambda i: (0, 0))
    g = pl.pallas_call(
        _make_transform_body(N),
        grid=(5,),
        in_specs=[pl.BlockSpec((bN, H), lambda i: (i, 0)),
                  nspec,
                  full2((8, H)),
                  full2((H, H)),
                  full2((1, H)),
                  full2((1, H)),
                  nspec, nspec, nspec, nspec],
        out_specs=pl.BlockSpec((bN, H), lambda i: (i, 0)),
        out_shape=jax.ShapeDtypeStruct((N, H), jnp.float32),
    )(x, node_att, stats, W, bn_weight.reshape(1, H), bn_bias.reshape(1, H),
      d0, d1, l0, l1)

    KTOT = -(-E // (NS * C2))
    KTOT += KTOT % 2
    KA = (3 * KTOT) // 8
    KA += KA % 2
    KB = KTOT - KA
    KMAX = max(KA, KB)
    EP2 = NS * KTOT * C2
    pk_flat = jnp.concatenate(
        [row | (col << 14), jnp.zeros((EP2 - E,), jnp.int32)])
    att_flat = jnp.concatenate(
        [edge_att, jnp.zeros((EP2 - E,), jnp.float32)])

    def _worker_layout(flat):
        pieces = []
        off = 0
        for t in range(NS):
            for kcnt in (KA, KB):
                n = kcnt * C2
                piece = lax.dynamic_slice(flat, (off,), (n,))
                piece = jnp.pad(piece, (0, KMAX * C2 - n))
                pieces.append(piece.reshape(1, KMAX, C2))
                off += n
        return jnp.concatenate(pieces, axis=0)

    pk_p = _worker_layout(pk_flat)
    att2_p = _worker_layout(att_flat)

    acc = _make_edge_kernel(N, NPAD, KA, KB, H)(pk_p, att2_p, g)

    aspec = pl.BlockSpec((NC, bN, H), lambda i: (0, i, 0))
    pooled = pl.pallas_call(
        _pool_body,
        grid=(5,),
        in_specs=[aspec,
                  pl.BlockSpec((bN, H), lambda i: (i, 0)),
                  nspec, nspec, nspec, nspec,
                  full2((1, H)),
                  pl.BlockSpec((bN, 1), lambda i: (i, 0))],
        out_specs=pl.BlockSpec((G, H), lambda i: (0, 0)),
        out_shape=jax.ShapeDtypeStruct((G, H), jnp.float32),
    )(acc, g, d0, d1, l0, l1,
      b.reshape(1, H), batch.reshape(N, 1))
    return pooled


# speedup vs baseline: 1.4117x; 1.0647x over previous
import functools

import jax
import jax.numpy as jnp
from jax import lax
from jax.experimental import pallas as pl
from jax.experimental.pallas import tpu as pltpu
from jax.experimental.pallas import tpu_sc as plsc

EPS = 1e-5
G = 256
NC, NS, L = 2, 16, 16
NW = NC * NS


def _mesh():
    return plsc.VectorSubcoreMesh(core_axis_name="c", subcore_axis_name="s")


def _make_deg_kernel(N, NPAD, KTOT):
    RPT = NPAD // NS
    KD = KTOT // NC

    @functools.partial(
        pl.kernel,
        out_type=(
            jax.ShapeDtypeStruct((NC, NPAD), jnp.float32),
            jax.ShapeDtypeStruct((NC, NPAD), jnp.float32),
        ),
        mesh=_mesh(),
        scratch_types=[
            pltpu.VMEM((KD, C2), jnp.int32),
            pltpu.VMEM((KD, C2), jnp.float32),
            pltpu.VMEM((NPAD,), jnp.float32),
            pltpu.VMEM((NPAD,), jnp.float32),
            pltpu.VMEM((NS, RPT), jnp.float32),
            pltpu.VMEM_SHARED((NS, NPAD), jnp.float32),
        ],
        compiler_params=pltpu.CompilerParams(
            needs_layout_passes=False, use_tc_tiling_on_sc=False),
    )
    def deg_kernel(pk_h, att_h, degp_h, lwp_h,
                   pkb, attb, degv, lwv, redb, sh):
        cid = lax.axis_index("c")
        tid = lax.axis_index("s")

        koff = cid * KD
        pltpu.sync_copy(pk_h.at[tid, pl.ds(koff, KD)], pkb)
        pltpu.sync_copy(att_h.at[tid, pl.ds(koff, KD)], attb)

        def init_body(i, _):
            degv[pl.ds(i * L, L)] = jnp.zeros((L,), jnp.float32)
            lwv[pl.ds(i * L, L)] = jnp.full((L,), -1.0, jnp.float32)
            return 0
        lax.fori_loop(0, NPAD // L, init_body, 0)

        def chunk_body(k, _):
            for j in range(C2 // L):
                p = pkb[k, pl.ds(j * L, L)]
                r = lax.bitwise_and(p, 16383)
                c = lax.shift_right_logical(p, 14)
                a = attb[k, pl.ds(j * L, L)]
                w = jnp.where(r != c, a, 0.0)
                plsc.addupdate_scatter(degv, [r], w)
                plsc.store_scatter(lwv, [r], a, mask=(r == c))
            return 0
        lax.fori_loop(0, KD, chunk_body, 0)

        base = tid * RPT
        pltpu.sync_copy(degv, sh.at[tid])
        plsc.subcore_barrier()
        pltpu.sync_copy(sh.at[:, pl.ds(base, RPT)], redb)

        def red_sum(v, _):
            acc = jnp.zeros((L,), jnp.float32)
            for rr in range(NS):
                acc = acc + redb[rr, pl.ds(v * L, L)]
            degv[pl.ds(v * L, L)] = acc
            return 0
        lax.fori_loop(0, RPT // L, red_sum, 0)
        pltpu.sync_copy(degv.at[pl.ds(0, RPT)], degp_h.at[cid, pl.ds(base, RPT)])
        plsc.subcore_barrier()

        pltpu.sync_copy(lwv, sh.at[tid])
        plsc.subcore_barrier()
        pltpu.sync_copy(sh.at[:, pl.ds(base, RPT)], redb)

        def red_max(v, _):
            acc = jnp.full((L,), -1.0, jnp.float32)
            for rr in range(NS):
                acc = jnp.maximum(acc, redb[rr, pl.ds(v * L, L)])
            lwv[pl.ds(v * L, L)] = acc
            return 0
        lax.fori_loop(0, RPT // L, red_max, 0)
        pltpu.sync_copy(lwv.at[pl.ds(0, RPT)], lwp_h.at[cid, pl.ds(base, RPT)])

    return deg_kernel


C2 = 80


def _make_edge_kernel(N, NPAD, KA, KB, H):
    RPT = NPAD // NS
    KMAX = max(KA, KB)

    @functools.partial(
        pl.kernel,
        out_type=jax.ShapeDtypeStruct((NC, NPAD, H), jnp.float32),
        mesh=_mesh(),
        scratch_types=[
            pltpu.VMEM((KMAX, C2), jnp.int32),
            pltpu.VMEM((2, C2), jnp.float32),
            pltpu.VMEM((2, C2), jnp.int32),
            pltpu.VMEM((2, C2), jnp.int32),
            pltpu.VMEM((2, C2), jnp.float32),
            pltpu.VMEM((C2, H), jnp.float32),
            pltpu.VMEM((C2, H), jnp.float32),
            pltpu.VMEM_SHARED((NPAD, H), jnp.float32),
            pltpu.SemaphoreType.DMA,
            pltpu.SemaphoreType.DMA,
            pltpu.SemaphoreType.DMA,
            pltpu.SemaphoreType.DMA,
        ],
        compiler_params=pltpu.CompilerParams(
            needs_layout_passes=False, use_tc_tiling_on_sc=False),
    )
    def edge_kernel(pk_h, att_h, g_h, acc_h,
                    pki, attb, rowi, coli, wbuf, gb0, gb1, acc_sh,
                    gs0, gs1, as0, as1):
        cid = lax.axis_index("c")
        tid = lax.axis_index("s")
        base = tid * RPT
        kw = jnp.where(cid == 0, KA, KB)
        koff = jnp.where(cid == 0, 0, KA)

        pltpu.sync_copy(pk_h.at[tid, pl.ds(koff, KMAX)], pki)

        asems = (as0, as1)

        def att_fetch(k, bb):
            pltpu.async_copy(
                att_h.at[tid, pl.ds(koff + k, 1)],
                attb.at[pl.ds(bb, 1)], asems[bb])

        def att_wait(k, bb):
            pltpu.make_async_copy(
                att_h.at[tid, pl.ds(koff + k, 1)],
                attb.at[pl.ds(bb, 1)], asems[bb]).wait()

        def unpack(k, bb):
            for j in range(C2 // L):
                p = pki[k, pl.ds(j * L, L)]
                r = lax.bitwise_and(p, 16383)
                c = lax.shift_right_logical(p, 14)
                rowi[bb, pl.ds(j * L, L)] = r
                coli[bb, pl.ds(j * L, L)] = c

        def z_body(i, _):
            for j in range(H // L):
                gb0[i, pl.ds(j * L, L)] = jnp.zeros((L,), jnp.float32)
            return 0
        lax.fori_loop(0, C2, z_body, 0)

        def zc_body(mb, _):
            off = pl.multiple_of(base + mb * C2, C2)
            pltpu.sync_copy(gb0, acc_sh.at[pl.ds(off, C2)])
            return 0
        lax.fori_loop(0, RPT // C2, zc_body, 0)
        plsc.subcore_barrier()

        gbs = (gb0, gb1)
        gsems = (gs0, gs1)

        unpack(0, 0)
        pltpu.async_copy(g_h.at[rowi.at[0]], gb0, gs0)
        att_fetch(0, 0)
        unpack(1, 1)
        pltpu.async_copy(g_h.at[rowi.at[1]], gb1, gs1)
        att_fetch(1, 1)

        def do_chunk(k, bb):
            gbuf, gsem = gbs[bb], gsems[bb]
            pltpu.make_async_copy(g_h.at[rowi.at[bb]], gbuf, gsem).wait()
            att_wait(k, bb)

            for j in range(C2 // L):
                r = rowi[bb, pl.ds(j * L, L)]
                c = coli[bb, pl.ds(j * L, L)]
                a = attb[bb, pl.ds(j * L, L)]
                wbuf[bb, pl.ds(j * L, L)] = jnp.where(r != c, a, 0.0)

            def e_body(e, _):
                ws = plsc.load_gather(
                    wbuf, [jnp.full((L,), bb, jnp.int32),
                           jnp.full((L,), e, jnp.int32)])
                for j in range(H // L):
                    gbuf[e, pl.ds(j * L, L)] = gbuf[e, pl.ds(j * L, L)] * ws
                return 0
            lax.fori_loop(0, C2, e_body, 0)

            pltpu.sync_copy(gbuf, acc_sh.at[coli.at[bb]], add=True)

            @pl.when(k + 2 < kw)
            def _():
                unpack(k + 2, bb)
                pltpu.async_copy(g_h.at[rowi.at[bb]], gbuf, gsem)
                att_fetch(k + 2, bb)

        def loop2(k2, _):
            for bb in range(2):
                do_chunk(k2 * 2 + bb, bb)
            return 0
        lax.fori_loop(0, kw // 2, loop2, 0)

        plsc.subcore_barrier()

        def rb_body(nb, _):
            off = pl.multiple_of(base + nb * 128, 128)
            pltpu.sync_copy(acc_sh.at[pl.ds(off, 128)],
                            acc_h.at[cid, pl.ds(off, 128)])
            return 0
        lax.fori_loop(0, RPT // 128, rb_body, 0)

    return edge_kernel


def _stats_body(x_ref, na_ref, o_ref):
    i = pl.program_id(0)
    xa = na_ref[...] * x_ref[...]

    @pl.when(i == 0)
    def _():
        o_ref[...] = jnp.zeros_like(o_ref)
    o_ref[0:1, :] += jnp.sum(xa, axis=0, keepdims=True)
    o_ref[1:2, :] += jnp.sum(xa * xa, axis=0, keepdims=True)


def _deg_terms(d0, d1, l0, l1):
    lwp = jnp.maximum(l0, l1)
    lw = jnp.where(lwp >= 0.0, lwp, 1.0)
    deg = d0 + d1 + lw
    dinv = jnp.where(deg > 0.0, lax.rsqrt(jnp.maximum(deg, EPS * EPS)), 0.0)
    return lw, dinv


def _make_transform_body(N):
    def body(x_ref, na_ref, st_ref, w_ref, bnw_ref, bnb_ref,
             d0_ref, d1_ref, l0_ref, l1_ref, g_ref):
        m = st_ref[0:1, :] * (1.0 / N)
        msq = st_ref[1:2, :] * (1.0 / N)
        var = msq - m * m
        s = bnw_ref[...] * lax.rsqrt(var + EPS)
        t = bnb_ref[...] - m * s
        xb = (na_ref[...] * x_ref[...]) * s + t
        h = jnp.dot(xb, w_ref[...], preferred_element_type=jnp.float32)
        _, dinv = _deg_terms(d0_ref[...], d1_ref[...], l0_ref[...], l1_ref[...])
        g_ref[...] = dinv * h
    return body


def _pool_body(acc_ref, g_ref,
               d0_ref, d1_ref, l0_ref, l1_ref, b_ref, bat_ref, o_ref):
    i = pl.program_id(0)
    lw, dinv = _deg_terms(d0_ref[...], d1_ref[...], l0_ref[...], l1_ref[...])
    acc = acc_ref[0] + acc_ref[1]
    out = dinv * (acc + lw * g_ref[...]) + b_ref[...]
    out = jnp.maximum(out, 0.0)
    oh = (bat_ref[...] == lax.broadcasted_iota(jnp.int32, (1, G), 1))
    p = lax.dot_general(oh.astype(jnp.float32), out,
                        (((0,), (0,)), ((), ())),
                        preferred_element_type=jnp.float32)

    @pl.when(i == 0)
    def _():
        o_ref[...] = jnp.zeros_like(o_ref)
    o_ref[...] += p


def kernel(x, edge_index, batch, edge_att, node_att, W, b, bn_weight, bn_bias):
    N, H = x.shape
    E = edge_att.shape[0]
    NPAD = -(-N // (NS * 128)) * (NS * 128)

    row = edge_index[0]
    col = edge_index[1]

    KTOT = -(-E // (NS * C2))
    KTOT += KTOT % 2
    KA = (29 * KTOT) // 100
    KA += KA % 2
    KB = KTOT - KA
    EP2 = NS * KTOT * C2
    pk_p = jnp.concatenate(
        [row | (col << 14), jnp.zeros((EP2 - E,), jnp.int32)]
    ).reshape(NS, KTOT, C2)
    att_p = jnp.concatenate(
        [edge_att, jnp.full((EP2 - E,), -1.0, jnp.float32)]
    ).reshape(NS, KTOT, C2)

    degp, lwp = _make_deg_kernel(N, NPAD, KTOT)(pk_p, att_p)

    bN = N // 5
    stats = pl.pallas_call(
        _stats_body,
        grid=(5,),
        in_specs=[pl.BlockSpec((bN, H), lambda i: (i, 0)),
                  pl.BlockSpec((bN, 1), lambda i: (i, 0))],
        out_specs=pl.BlockSpec((8, H), lambda i: (0, 0)),
        out_shape=jax.ShapeDtypeStruct((8, H), jnp.float32),
    )(x, node_att)

    d0 = degp[0, :N].reshape(N, 1)
    d1 = degp[1, :N].reshape(N, 1)
    l0 = lwp[0, :N].reshape(N, 1)
    l1 = lwp[1, :N].reshape(N, 1)

    nspec = pl.BlockSpec((bN, 1), lambda i: (i, 0))
    full2 = lambda shape: pl.BlockSpec(shape, lambda i: (0, 0))
    g = pl.pallas_call(
        _make_transform_body(N),
        grid=(5,),
        in_specs=[pl.BlockSpec((bN, H), lambda i: (i, 0)),
                  nspec,
                  full2((8, H)),
                  full2((H, H)),
                  full2((1, H)),
                  full2((1, H)),
                  nspec, nspec, nspec, nspec],
        out_specs=pl.BlockSpec((bN, H), lambda i: (i, 0)),
        out_shape=jax.ShapeDtypeStruct((N, H), jnp.float32),
    )(x, node_att, stats, W, bn_weight.reshape(1, H), bn_bias.reshape(1, H),
      d0, d1, l0, l1)

    acc = _make_edge_kernel(N, NPAD, KA, KB, H)(pk_p, att_p, g)

    aspec = pl.BlockSpec((NC, bN, H), lambda i: (0, i, 0))
    pooled = pl.pallas_call(
        _pool_body,
        grid=(5,),
        in_specs=[aspec,
                  pl.BlockSpec((bN, H), lambda i: (i, 0)),
                  nspec, nspec, nspec, nspec,
                  full2((1, H)),
                  pl.BlockSpec((bN, 1), lambda i: (i, 0))],
        out_specs=pl.BlockSpec((G, H), lambda i: (0, 0)),
        out_shape=jax.ShapeDtypeStruct((G, H), jnp.float32),
    )(acc, g, d0, d1, l0, l1,
      b.reshape(1, H), batch.reshape(N, 1))
    return pooled


# speedup vs baseline: 1.6104x; 1.1408x over previous
import functools

import jax
import jax.numpy as jnp
from jax import lax
from jax.experimental import pallas as pl
from jax.experimental.pallas import tpu as pltpu
from jax.experimental.pallas import tpu_sc as plsc

EPS = 1e-5
G = 256
NC, NS, L = 2, 16, 16
NW = NC * NS


def _mesh():
    return plsc.VectorSubcoreMesh(core_axis_name="c", subcore_axis_name="s")


def _make_deg_kernel(N, NPAD, KTOT):
    RPT = NPAD // NS
    KD = KTOT // NC

    @functools.partial(
        pl.kernel,
        out_type=(
            jax.ShapeDtypeStruct((NC, NPAD), jnp.float32),
            jax.ShapeDtypeStruct((NC, NPAD), jnp.float32),
        ),
        mesh=_mesh(),
        scratch_types=[
            pltpu.VMEM((KD, C2), jnp.int32),
            pltpu.VMEM((KD, C2), jnp.float32),
            pltpu.VMEM((NPAD,), jnp.float32),
            pltpu.VMEM((NPAD,), jnp.float32),
            pltpu.VMEM((NS, RPT), jnp.float32),
            pltpu.VMEM_SHARED((NS, NPAD), jnp.float32),
        ],
        compiler_params=pltpu.CompilerParams(
            needs_layout_passes=False, use_tc_tiling_on_sc=False),
    )
    def deg_kernel(pk_h, att_h, degp_h, lwp_h,
                   pkb, attb, degv, lwv, redb, sh):
        cid = lax.axis_index("c")
        tid = lax.axis_index("s")

        koff = cid * KD
        pltpu.sync_copy(pk_h.at[tid, pl.ds(koff, KD)], pkb)
        pltpu.sync_copy(att_h.at[tid, pl.ds(koff, KD)], attb)

        def init_body(i, _):
            degv[pl.ds(i * L, L)] = jnp.zeros((L,), jnp.float32)
            lwv[pl.ds(i * L, L)] = jnp.full((L,), -1.0, jnp.float32)
            return 0
        lax.fori_loop(0, NPAD // L, init_body, 0)

        def chunk_body(k, _):
            for j in range(C2 // L):
                p = pkb[k, pl.ds(j * L, L)]
                r = lax.bitwise_and(p, 16383)
                c = lax.shift_right_logical(p, 14)
                a = attb[k, pl.ds(j * L, L)]
                w = jnp.where(r != c, a, 0.0)
                plsc.addupdate_scatter(degv, [r], w)
                plsc.store_scatter(lwv, [r], a, mask=(r == c))
            return 0
        lax.fori_loop(0, KD, chunk_body, 0)

        base = tid * RPT
        pltpu.sync_copy(degv, sh.at[tid])
        plsc.subcore_barrier()
        pltpu.sync_copy(sh.at[:, pl.ds(base, RPT)], redb)

        def red_sum(v, _):
            acc = jnp.zeros((L,), jnp.float32)
            for rr in range(NS):
                acc = acc + redb[rr, pl.ds(v * L, L)]
            degv[pl.ds(v * L, L)] = acc
            return 0
        lax.fori_loop(0, RPT // L, red_sum, 0)
        pltpu.sync_copy(degv.at[pl.ds(0, RPT)], degp_h.at[cid, pl.ds(base, RPT)])
        plsc.subcore_barrier()

        pltpu.sync_copy(lwv, sh.at[tid])
        plsc.subcore_barrier()
        pltpu.sync_copy(sh.at[:, pl.ds(base, RPT)], redb)

        def red_max(v, _):
            acc = jnp.full((L,), -1.0, jnp.float32)
            for rr in range(NS):
                acc = jnp.maximum(acc, redb[rr, pl.ds(v * L, L)])
            lwv[pl.ds(v * L, L)] = acc
            return 0
        lax.fori_loop(0, RPT // L, red_max, 0)
        pltpu.sync_copy(lwv.at[pl.ds(0, RPT)], lwp_h.at[cid, pl.ds(base, RPT)])

    return deg_kernel


C2 = 80


def _make_edge_kernel(N, NPAD, KA, KB, H):
    RPT = NPAD // NS
    KMAX = max(KA, KB)

    @functools.partial(
        pl.kernel,
        out_type=jax.ShapeDtypeStruct((NC, NPAD, H), jnp.float32),
        mesh=_mesh(),
        scratch_types=[
            pltpu.VMEM((KMAX, C2), jnp.int32),
            pltpu.VMEM((2, C2), jnp.float32),
            pltpu.VMEM((2, C2), jnp.int32),
            pltpu.VMEM((2, C2), jnp.int32),
            pltpu.VMEM((2, C2), jnp.float32),
            pltpu.VMEM((C2, H), jnp.float32),
            pltpu.VMEM((C2, H), jnp.float32),
            pltpu.VMEM_SHARED((NPAD, H), jnp.float32),
            pltpu.SemaphoreType.DMA,
            pltpu.SemaphoreType.DMA,
            pltpu.SemaphoreType.DMA,
            pltpu.SemaphoreType.DMA,
        ],
        compiler_params=pltpu.CompilerParams(
            needs_layout_passes=False, use_tc_tiling_on_sc=False),
    )
    def edge_kernel(pk_h, att_h, g_h, acc_h,
                    pki, attb, rowi, coli, wbuf, gb0, gb1, acc_sh,
                    gs0, gs1, as0, as1):
        cid = lax.axis_index("c")
        tid = lax.axis_index("s")
        base = tid * RPT
        kw = jnp.where(cid == 0, KA, KB)
        koff = jnp.where(cid == 0, 0, KA)

        pltpu.sync_copy(pk_h.at[tid, pl.ds(koff, KMAX)], pki)

        asems = (as0, as1)

        def att_fetch(k, bb):
            pltpu.async_copy(
                att_h.at[tid, pl.ds(koff + k, 1)],
                attb.at[pl.ds(bb, 1)], asems[bb])

        def att_wait(k, bb):
            pltpu.make_async_copy(
                att_h.at[tid, pl.ds(koff + k, 1)],
                attb.at[pl.ds(bb, 1)], asems[bb]).wait()

        def unpack(k, bb):
            for j in range(C2 // L):
                p = pki[k, pl.ds(j * L, L)]
                r = lax.bitwise_and(p, 16383)
                c = lax.shift_right_logical(p, 14)
                rowi[bb, pl.ds(j * L, L)] = r
                coli[bb, pl.ds(j * L, L)] = c

        def z_body(i, _):
            for j in range(H // L):
                gb0[i, pl.ds(j * L, L)] = jnp.zeros((L,), jnp.float32)
            return 0
        lax.fori_loop(0, C2, z_body, 0)

        def zc_body(mb, _):
            off = pl.multiple_of(base + mb * C2, C2)
            pltpu.sync_copy(gb0, acc_sh.at[pl.ds(off, C2)])
            return 0
        lax.fori_loop(0, RPT // C2, zc_body, 0)
        plsc.subcore_barrier()

        gbs = (gb0, gb1)
        gsems = (gs0, gs1)

        unpack(0, 0)
        pltpu.async_copy(g_h.at[rowi.at[0]], gb0, gs0)
        att_fetch(0, 0)
        unpack(1, 1)
        pltpu.async_copy(g_h.at[rowi.at[1]], gb1, gs1)
        att_fetch(1, 1)

        def do_chunk(k, bb):
            gbuf, gsem = gbs[bb], gsems[bb]
            pltpu.make_async_copy(g_h.at[rowi.at[bb]], gbuf, gsem).wait()
            att_wait(k, bb)

            for j in range(C2 // L):
                r = rowi[bb, pl.ds(j * L, L)]
                c = coli[bb, pl.ds(j * L, L)]
                a = attb[bb, pl.ds(j * L, L)]
                wbuf[bb, pl.ds(j * L, L)] = jnp.where(r != c, a, 0.0)

            def e_body(e, _):
                ws = plsc.load_gather(
                    wbuf, [jnp.full((L,), bb, jnp.int32),
                           jnp.full((L,), e, jnp.int32)])
                for j in range(H // L):
                    gbuf[e, pl.ds(j * L, L)] = gbuf[e, pl.ds(j * L, L)] * ws
                return 0
            lax.fori_loop(0, C2, e_body, 0)

            pltpu.sync_copy(gbuf, acc_sh.at[coli.at[bb]], add=True)

            @pl.when(k + 2 < kw)
            def _():
                unpack(k + 2, bb)
                pltpu.async_copy(g_h.at[rowi.at[bb]], gbuf, gsem)
                att_fetch(k + 2, bb)

        def loop2(k2, _):
            for bb in range(2):
                do_chunk(k2 * 2 + bb, bb)
            return 0
        lax.fori_loop(0, kw // 2, loop2, 0)

        plsc.subcore_barrier()

        def rb_body(nb, _):
            off = pl.multiple_of(base + nb * 128, 128)
            pltpu.sync_copy(acc_sh.at[pl.ds(off, 128)],
                            acc_h.at[cid, pl.ds(off, 128)])
            return 0
        lax.fori_loop(0, RPT // 128, rb_body, 0)

    return edge_kernel


def _stats_body(x_ref, na_ref, o_ref):
    i = pl.program_id(0)
    xa = na_ref[...] * x_ref[...]

    @pl.when(i == 0)
    def _():
        o_ref[...] = jnp.zeros_like(o_ref)
    o_ref[0:1, :] += jnp.sum(xa, axis=0, keepdims=True)
    o_ref[1:2, :] += jnp.sum(xa * xa, axis=0, keepdims=True)


def _deg_terms(d0, d1, l0, l1):
    lwp = jnp.maximum(l0, l1)
    lw = jnp.where(lwp >= 0.0, lwp, 1.0)
    deg = d0 + d1 + lw
    dinv = jnp.where(deg > 0.0, lax.rsqrt(jnp.maximum(deg, EPS * EPS)), 0.0)
    return lw, dinv


def _make_transform_body(N):
    def body(x_ref, na_ref, st_ref, w_ref, bnw_ref, bnb_ref,
             d0_ref, d1_ref, l0_ref, l1_ref, g_ref):
        m = st_ref[0:1, :] * (1.0 / N)
        msq = st_ref[1:2, :] * (1.0 / N)
        var = msq - m * m
        s = bnw_ref[...] * lax.rsqrt(var + EPS)
        t = bnb_ref[...] - m * s
        xb = (na_ref[...] * x_ref[...]) * s + t
        h = jnp.dot(xb, w_ref[...], preferred_element_type=jnp.float32)
        _, dinv = _deg_terms(d0_ref[...], d1_ref[...], l0_ref[...], l1_ref[...])
        g_ref[...] = dinv * h
    return body


def _pool_body(acc_ref, g_ref,
               d0_ref, d1_ref, l0_ref, l1_ref, b_ref, bat_ref, o_ref):
    i = pl.program_id(0)
    lw, dinv = _deg_terms(d0_ref[...], d1_ref[...], l0_ref[...], l1_ref[...])
    acc = acc_ref[0] + acc_ref[1]
    out = dinv * (acc + lw * g_ref[...]) + b_ref[...]
    out = jnp.maximum(out, 0.0)
    oh = (bat_ref[...] == lax.broadcasted_iota(jnp.int32, (1, G), 1))
    p = lax.dot_general(oh.astype(jnp.float32), out,
                        (((0,), (0,)), ((), ())),
                        preferred_element_type=jnp.float32)

    @pl.when(i == 0)
    def _():
        o_ref[...] = jnp.zeros_like(o_ref)
    o_ref[...] += p


def kernel(x, edge_index, batch, edge_att, node_att, W, b, bn_weight, bn_bias):
    N, H = x.shape
    E = edge_att.shape[0]
    NPAD = -(-N // (NS * 128)) * (NS * 128)

    row = edge_index[0]
    col = edge_index[1]

    KTOT = -(-E // (NS * C2))
    KTOT += KTOT % 2
    KA = (415 * KTOT) // 1000
    KA += KA % 2
    KB = KTOT - KA
    EP2 = NS * KTOT * C2
    pk_p = jnp.concatenate(
        [row | (col << 14), jnp.zeros((EP2 - E,), jnp.int32)]
    ).reshape(NS, KTOT, C2)
    att_p = jnp.concatenate(
        [edge_att, jnp.full((EP2 - E,), -1.0, jnp.float32)]
    ).reshape(NS, KTOT, C2)

    degp, lwp = _make_deg_kernel(N, NPAD, KTOT)(pk_p, att_p)

    bN = N // 5
    stats = pl.pallas_call(
        _stats_body,
        grid=(5,),
        in_specs=[pl.BlockSpec((bN, H), lambda i: (i, 0)),
                  pl.BlockSpec((bN, 1), lambda i: (i, 0))],
        out_specs=pl.BlockSpec((8, H), lambda i: (0, 0)),
        out_shape=jax.ShapeDtypeStruct((8, H), jnp.float32),
    )(x, node_att)

    d0 = degp[0, :N].reshape(N, 1)
    d1 = degp[1, :N].reshape(N, 1)
    l0 = lwp[0, :N].reshape(N, 1)
    l1 = lwp[1, :N].reshape(N, 1)

    nspec = pl.BlockSpec((bN, 1), lambda i: (i, 0))
    full2 = lambda shape: pl.BlockSpec(shape, lambda i: (0, 0))
    g = pl.pallas_call(
        _make_transform_body(N),
        grid=(5,),
        in_specs=[pl.BlockSpec((bN, H), lambda i: (i, 0)),
                  nspec,
                  full2((8, H)),
                  full2((H, H)),
                  full2((1, H)),
                  full2((1, H)),
                  nspec, nspec, nspec, nspec],
        out_specs=pl.BlockSpec((bN, H), lambda i: (i, 0)),
        out_shape=jax.ShapeDtypeStruct((N, H), jnp.float32),
    )(x, node_att, stats, W, bn_weight.reshape(1, H), bn_bias.reshape(1, H),
      d0, d1, l0, l1)

    acc = _make_edge_kernel(N, NPAD, KA, KB, H)(pk_p, att_p, g)

    aspec = pl.BlockSpec((NC, bN, H), lambda i: (0, i, 0))
    pooled = pl.pallas_call(
        _pool_body,
        grid=(5,),
        in_specs=[aspec,
                  pl.BlockSpec((bN, H), lambda i: (i, 0)),
                  nspec, nspec, nspec, nspec,
                  full2((1, H)),
                  pl.BlockSpec((bN, 1), lambda i: (i, 0))],
        out_specs=pl.BlockSpec((G, H), lambda i: (0, 0)),
        out_shape=jax.ShapeDtypeStruct((G, H), jnp.float32),
    )(acc, g, d0, d1, l0, l1,
      b.reshape(1, H), batch.reshape(N, 1))
    return pooled


# speedup vs baseline: 1.7023x; 1.0571x over previous
import functools

import jax
import jax.numpy as jnp
from jax import lax
from jax.experimental import pallas as pl
from jax.experimental.pallas import tpu as pltpu
from jax.experimental.pallas import tpu_sc as plsc

EPS = 1e-5
G = 256
NC, NS, L = 2, 16, 16
NW = NC * NS


def _mesh():
    return plsc.VectorSubcoreMesh(core_axis_name="c", subcore_axis_name="s")


def _make_deg_kernel(N, NPAD, KTOT):
    RPT = NPAD // NS
    KD = KTOT // NC

    @functools.partial(
        pl.kernel,
        out_type=(
            jax.ShapeDtypeStruct((NC, NPAD), jnp.float32),
            jax.ShapeDtypeStruct((NC, NPAD), jnp.float32),
        ),
        mesh=_mesh(),
        scratch_types=[
            pltpu.VMEM((KD, C2), jnp.int32),
            pltpu.VMEM((KD, C2), jnp.float32),
            pltpu.VMEM((NPAD,), jnp.float32),
            pltpu.VMEM((NPAD,), jnp.float32),
            pltpu.VMEM((NS, RPT), jnp.float32),
            pltpu.VMEM_SHARED((NS, NPAD), jnp.float32),
        ],
        compiler_params=pltpu.CompilerParams(
            needs_layout_passes=False, use_tc_tiling_on_sc=False),
    )
    def deg_kernel(pk_h, att_h, degp_h, lwp_h,
                   pkb, attb, degv, lwv, redb, sh):
        cid = lax.axis_index("c")
        tid = lax.axis_index("s")

        koff = cid * KD
        pltpu.sync_copy(pk_h.at[tid, pl.ds(koff, KD)], pkb)
        pltpu.sync_copy(att_h.at[tid, pl.ds(koff, KD)], attb)

        def init_body(i, _):
            degv[pl.ds(i * L, L)] = jnp.zeros((L,), jnp.float32)
            lwv[pl.ds(i * L, L)] = jnp.full((L,), -1.0, jnp.float32)
            return 0
        lax.fori_loop(0, NPAD // L, init_body, 0)

        def chunk_body(k, _):
            for j in range(C2 // L):
                p = pkb[k, pl.ds(j * L, L)]
                r = lax.bitwise_and(p, 16383)
                c = lax.shift_right_logical(p, 14)
                a = attb[k, pl.ds(j * L, L)]
                w = jnp.where(r != c, a, 0.0)
                plsc.addupdate_scatter(degv, [r], w)
                plsc.store_scatter(lwv, [r], a, mask=(r == c))
            return 0
        lax.fori_loop(0, KD, chunk_body, 0)

        base = tid * RPT
        pltpu.sync_copy(degv, sh.at[tid])
        plsc.subcore_barrier()
        pltpu.sync_copy(sh.at[:, pl.ds(base, RPT)], redb)

        def red_sum(v, _):
            acc = jnp.zeros((L,), jnp.float32)
            for rr in range(NS):
                acc = acc + redb[rr, pl.ds(v * L, L)]
            degv[pl.ds(v * L, L)] = acc
            return 0
        lax.fori_loop(0, RPT // L, red_sum, 0)
        pltpu.sync_copy(degv.at[pl.ds(0, RPT)], degp_h.at[cid, pl.ds(base, RPT)])
        plsc.subcore_barrier()

        pltpu.sync_copy(lwv, sh.at[tid])
        plsc.subcore_barrier()
        pltpu.sync_copy(sh.at[:, pl.ds(base, RPT)], redb)

        def red_max(v, _):
            acc = jnp.full((L,), -1.0, jnp.float32)
            for rr in range(NS):
                acc = jnp.maximum(acc, redb[rr, pl.ds(v * L, L)])
            lwv[pl.ds(v * L, L)] = acc
            return 0
        lax.fori_loop(0, RPT // L, red_max, 0)
        pltpu.sync_copy(lwv.at[pl.ds(0, RPT)], lwp_h.at[cid, pl.ds(base, RPT)])

    return deg_kernel


C2 = 80


def _make_edge_kernel(N, NPAD, KA, KB, H):
    RPT = NPAD // NS
    KMAX = max(KA, KB)

    @functools.partial(
        pl.kernel,
        out_type=jax.ShapeDtypeStruct((NC, NPAD, H), jnp.float32),
        mesh=_mesh(),
        scratch_types=[
            pltpu.VMEM((KMAX, C2), jnp.int32),
            pltpu.VMEM((2, C2), jnp.float32),
            pltpu.VMEM((2, C2), jnp.int32),
            pltpu.VMEM((2, C2), jnp.int32),
            pltpu.VMEM((2, C2), jnp.float32),
            pltpu.VMEM((C2, H), jnp.float32),
            pltpu.VMEM((C2, H), jnp.float32),
            pltpu.VMEM_SHARED((NPAD, H), jnp.float32),
            pltpu.SemaphoreType.DMA,
            pltpu.SemaphoreType.DMA,
            pltpu.SemaphoreType.DMA,
            pltpu.SemaphoreType.DMA,
        ],
        compiler_params=pltpu.CompilerParams(
            needs_layout_passes=False, use_tc_tiling_on_sc=False),
    )
    def edge_kernel(pk_h, att_h, g_h, acc_h,
                    pki, attb, rowi, coli, wbuf, gb0, gb1, acc_sh,
                    gs0, gs1, as0, as1):
        cid = lax.axis_index("c")
        tid = lax.axis_index("s")
        base = tid * RPT
        kw = jnp.where(cid == 0, KA, KB)
        koff = jnp.where(cid == 0, 0, KA)

        pltpu.sync_copy(pk_h.at[tid, pl.ds(koff, KMAX)], pki)

        asems = (as0, as1)

        def att_fetch(k, bb):
            pltpu.async_copy(
                att_h.at[tid, pl.ds(koff + k, 1)],
                attb.at[pl.ds(bb, 1)], asems[bb])

        def att_wait(k, bb):
            pltpu.make_async_copy(
                att_h.at[tid, pl.ds(koff + k, 1)],
                attb.at[pl.ds(bb, 1)], asems[bb]).wait()

        def unpack(k, bb):
            for j in range(C2 // L):
                p = pki[k, pl.ds(j * L, L)]
                r = lax.bitwise_and(p, 16383)
                c = lax.shift_right_logical(p, 14)
                rowi[bb, pl.ds(j * L, L)] = r
                coli[bb, pl.ds(j * L, L)] = c

        def z_body(i, _):
            for j in range(H // L):
                gb0[i, pl.ds(j * L, L)] = jnp.zeros((L,), jnp.float32)
            return 0
        lax.fori_loop(0, C2, z_body, 0)

        def zc_body(mb, _):
            off = pl.multiple_of(base + mb * C2, C2)
            pltpu.sync_copy(gb0, acc_sh.at[pl.ds(off, C2)])
            return 0
        lax.fori_loop(0, RPT // C2, zc_body, 0)
        plsc.subcore_barrier()

        gbs = (gb0, gb1)
        gsems = (gs0, gs1)

        unpack(0, 0)
        pltpu.async_copy(g_h.at[rowi.at[0]], gb0, gs0)
        att_fetch(0, 0)
        unpack(1, 1)
        pltpu.async_copy(g_h.at[rowi.at[1]], gb1, gs1)
        att_fetch(1, 1)

        def do_chunk(k, bb):
            gbuf, gsem = gbs[bb], gsems[bb]
            pltpu.make_async_copy(g_h.at[rowi.at[bb]], gbuf, gsem).wait()
            att_wait(k, bb)

            for j in range(C2 // L):
                r = rowi[bb, pl.ds(j * L, L)]
                c = coli[bb, pl.ds(j * L, L)]
                a = attb[bb, pl.ds(j * L, L)]
                wbuf[bb, pl.ds(j * L, L)] = jnp.where(r != c, a, 0.0)

            def e_body(e, _):
                ws = plsc.load_gather(
                    wbuf, [jnp.full((L,), bb, jnp.int32),
                           jnp.full((L,), e, jnp.int32)])
                for j in range(H // L):
                    gbuf[e, pl.ds(j * L, L)] = gbuf[e, pl.ds(j * L, L)] * ws
                return 0
            lax.fori_loop(0, C2, e_body, 0)

            pltpu.sync_copy(gbuf, acc_sh.at[coli.at[bb]], add=True)

            @pl.when(k + 2 < kw)
            def _():
                unpack(k + 2, bb)
                pltpu.async_copy(g_h.at[rowi.at[bb]], gbuf, gsem)
                att_fetch(k + 2, bb)

        def loop2(k2, _):
            for bb in range(2):
                do_chunk(k2 * 2 + bb, bb)
            return 0
        lax.fori_loop(0, kw // 2, loop2, 0)

        plsc.subcore_barrier()

        def rb_body(nb, _):
            off = pl.multiple_of(base + nb * 128, 128)
            pltpu.sync_copy(acc_sh.at[pl.ds(off, 128)],
                            acc_h.at[cid, pl.ds(off, 128)])
            return 0
        lax.fori_loop(0, RPT // 128, rb_body, 0)

    return edge_kernel


def _stats_body(x_ref, na_ref, o_ref):
    i = pl.program_id(0)
    xa = na_ref[...] * x_ref[...]

    @pl.when(i == 0)
    def _():
        o_ref[...] = jnp.zeros_like(o_ref)
    o_ref[0:1, :] += jnp.sum(xa, axis=0, keepdims=True)
    o_ref[1:2, :] += jnp.sum(xa * xa, axis=0, keepdims=True)


def _deg_terms(d0, d1, l0, l1):
    lwp = jnp.maximum(l0, l1)
    lw = jnp.where(lwp >= 0.0, lwp, 1.0)
    deg = d0 + d1 + lw
    dinv = jnp.where(deg > 0.0, lax.rsqrt(jnp.maximum(deg, EPS * EPS)), 0.0)
    return lw, dinv


def _make_transform_body(N):
    def body(x_ref, na_ref, st_ref, w_ref, bnw_ref, bnb_ref,
             d0_ref, d1_ref, l0_ref, l1_ref, g_ref):
        m = st_ref[0:1, :] * (1.0 / N)
        msq = st_ref[1:2, :] * (1.0 / N)
        var = msq - m * m
        s = bnw_ref[...] * lax.rsqrt(var + EPS)
        t = bnb_ref[...] - m * s
        xb = (na_ref[...] * x_ref[...]) * s + t
        h = jnp.dot(xb, w_ref[...], preferred_element_type=jnp.float32)
        _, dinv = _deg_terms(d0_ref[...], d1_ref[...], l0_ref[...], l1_ref[...])
        g_ref[...] = dinv * h
    return body


def _pool_body(acc_ref, g_ref,
               d0_ref, d1_ref, l0_ref, l1_ref, b_ref, bat_ref, o_ref):
    i = pl.program_id(0)
    lw, dinv = _deg_terms(d0_ref[...], d1_ref[...], l0_ref[...], l1_ref[...])
    acc = acc_ref[0] + acc_ref[1]
    out = dinv * (acc + lw * g_ref[...]) + b_ref[...]
    out = jnp.maximum(out, 0.0)
    oh = (bat_ref[...] == lax.broadcasted_iota(jnp.int32, (1, G), 1))
    p = lax.dot_general(oh.astype(jnp.float32), out,
                        (((0,), (0,)), ((), ())),
                        preferred_element_type=jnp.float32)

    @pl.when(i == 0)
    def _():
        o_ref[...] = jnp.zeros_like(o_ref)
    o_ref[...] += p


def kernel(x, edge_index, batch, edge_att, node_att, W, b, bn_weight, bn_bias):
    N, H = x.shape
    E = edge_att.shape[0]
    NPAD = -(-N // (NS * 128)) * (NS * 128)

    row = edge_index[0]
    col = edge_index[1]

    KTOT = -(-E // (NS * C2))
    KTOT += KTOT % 2
    KA = (465 * KTOT) // 1000
    KA += KA % 2
    KB = KTOT - KA
    EP2 = NS * KTOT * C2
    pk_p = jnp.concatenate(
        [row | (col << 14), jnp.zeros((EP2 - E,), jnp.int32)]
    ).reshape(NS, KTOT, C2)
    att_p = jnp.concatenate(
        [edge_att, jnp.full((EP2 - E,), -1.0, jnp.float32)]
    ).reshape(NS, KTOT, C2)

    degp, lwp = _make_deg_kernel(N, NPAD, KTOT)(pk_p, att_p)

    bN = N // 5
    stats = pl.pallas_call(
        _stats_body,
        grid=(5,),
        in_specs=[pl.BlockSpec((bN, H), lambda i: (i, 0)),
                  pl.BlockSpec((bN, 1), lambda i: (i, 0))],
        out_specs=pl.BlockSpec((8, H), lambda i: (0, 0)),
        out_shape=jax.ShapeDtypeStruct((8, H), jnp.float32),
    )(x, node_att)

    d0 = degp[0, :N].reshape(N, 1)
    d1 = degp[1, :N].reshape(N, 1)
    l0 = lwp[0, :N].reshape(N, 1)
    l1 = lwp[1, :N].reshape(N, 1)

    nspec = pl.BlockSpec((bN, 1), lambda i: (i, 0))
    full2 = lambda shape: pl.BlockSpec(shape, lambda i: (0, 0))
    g = pl.pallas_call(
        _make_transform_body(N),
        grid=(5,),
        in_specs=[pl.BlockSpec((bN, H), lambda i: (i, 0)),
                  nspec,
                  full2((8, H)),
                  full2((H, H)),
                  full2((1, H)),
                  full2((1, H)),
                  nspec, nspec, nspec, nspec],
        out_specs=pl.BlockSpec((bN, H), lambda i: (i, 0)),
        out_shape=jax.ShapeDtypeStruct((N, H), jnp.float32),
    )(x, node_att, stats, W, bn_weight.reshape(1, H), bn_bias.reshape(1, H),
      d0, d1, l0, l1)

    acc = _make_edge_kernel(N, NPAD, KA, KB, H)(pk_p, att_p, g)

    aspec = pl.BlockSpec((NC, bN, H), lambda i: (0, i, 0))
    pooled = pl.pallas_call(
        _pool_body,
        grid=(5,),
        in_specs=[aspec,
                  pl.BlockSpec((bN, H), lambda i: (i, 0)),
                  nspec, nspec, nspec, nspec,
                  full2((1, H)),
                  pl.BlockSpec((bN, 1), lambda i: (i, 0))],
        out_specs=pl.BlockSpec((G, H), lambda i: (0, 0)),
        out_shape=jax.ShapeDtypeStruct((G, H), jnp.float32),
    )(acc, g, d0, d1, l0, l1,
      b.reshape(1, H), batch.reshape(N, 1))
    return pooled


# speedup vs baseline: 1.7682x; 1.0387x over previous
import functools

import jax
import jax.numpy as jnp
from jax import lax
from jax.experimental import pallas as pl
from jax.experimental.pallas import tpu as pltpu
from jax.experimental.pallas import tpu_sc as plsc

EPS = 1e-5
G = 256
NC, NS, L = 2, 16, 16
NW = NC * NS


def _mesh():
    return plsc.VectorSubcoreMesh(core_axis_name="c", subcore_axis_name="s")


def _make_deg_kernel(N, NPAD, KTOT):
    RPT = NPAD // NS
    KD = KTOT // NC

    @functools.partial(
        pl.kernel,
        out_type=(
            jax.ShapeDtypeStruct((NC, NPAD), jnp.float32),
            jax.ShapeDtypeStruct((NC, NPAD), jnp.float32),
        ),
        mesh=_mesh(),
        scratch_types=[
            pltpu.VMEM((KD, C2), jnp.int32),
            pltpu.VMEM((KD, C2), jnp.float32),
            pltpu.VMEM((NPAD,), jnp.float32),
            pltpu.VMEM((NPAD,), jnp.float32),
            pltpu.VMEM((NS, RPT), jnp.float32),
            pltpu.VMEM_SHARED((NS, NPAD), jnp.float32),
        ],
        compiler_params=pltpu.CompilerParams(
            needs_layout_passes=False, use_tc_tiling_on_sc=False),
    )
    def deg_kernel(pk_h, att_h, degp_h, lwp_h,
                   pkb, attb, degv, lwv, redb, sh):
        cid = lax.axis_index("c")
        tid = lax.axis_index("s")

        koff = cid * KD
        pltpu.sync_copy(pk_h.at[tid, pl.ds(koff, KD)], pkb)
        pltpu.sync_copy(att_h.at[tid, pl.ds(koff, KD)], attb)

        def init_body(i, _):
            degv[pl.ds(i * L, L)] = jnp.zeros((L,), jnp.float32)
            lwv[pl.ds(i * L, L)] = jnp.full((L,), -1.0, jnp.float32)
            return 0
        lax.fori_loop(0, NPAD // L, init_body, 0)

        def chunk_body(k, _):
            for j in range(C2 // L):
                p = pkb[k, pl.ds(j * L, L)]
                r = lax.bitwise_and(p, 16383)
                c = lax.shift_right_logical(p, 14)
                a = attb[k, pl.ds(j * L, L)]
                w = jnp.where(r != c, a, 0.0)
                plsc.addupdate_scatter(degv, [r], w)
                plsc.store_scatter(lwv, [r], a, mask=(r == c))
            return 0
        lax.fori_loop(0, KD, chunk_body, 0)

        base = tid * RPT
        pltpu.sync_copy(degv, sh.at[tid])
        plsc.subcore_barrier()
        pltpu.sync_copy(sh.at[:, pl.ds(base, RPT)], redb)

        def red_sum(v, _):
            acc = jnp.zeros((L,), jnp.float32)
            for rr in range(NS):
                acc = acc + redb[rr, pl.ds(v * L, L)]
            degv[pl.ds(v * L, L)] = acc
            return 0
        lax.fori_loop(0, RPT // L, red_sum, 0)
        pltpu.sync_copy(degv.at[pl.ds(0, RPT)], degp_h.at[cid, pl.ds(base, RPT)])
        plsc.subcore_barrier()

        pltpu.sync_copy(lwv, sh.at[tid])
        plsc.subcore_barrier()
        pltpu.sync_copy(sh.at[:, pl.ds(base, RPT)], redb)

        def red_max(v, _):
            acc = jnp.full((L,), -1.0, jnp.float32)
            for rr in range(NS):
                acc = jnp.maximum(acc, redb[rr, pl.ds(v * L, L)])
            lwv[pl.ds(v * L, L)] = acc
            return 0
        lax.fori_loop(0, RPT // L, red_max, 0)
        pltpu.sync_copy(lwv.at[pl.ds(0, RPT)], lwp_h.at[cid, pl.ds(base, RPT)])

    return deg_kernel


C2 = 80


def _make_edge_kernel(N, NPAD, KA, KB, H):
    RPT = NPAD // NS
    KMAX = max(KA, KB)

    @functools.partial(
        pl.kernel,
        out_type=jax.ShapeDtypeStruct((NC, NPAD, H), jnp.float32),
        mesh=_mesh(),
        scratch_types=[
            pltpu.VMEM((KMAX, C2), jnp.int32),
            pltpu.VMEM((2, C2), jnp.float32),
            pltpu.VMEM((2, C2), jnp.int32),
            pltpu.VMEM((2, C2), jnp.int32),
            pltpu.VMEM((2, C2), jnp.float32),
            pltpu.VMEM((C2, H), jnp.float32),
            pltpu.VMEM((C2, H), jnp.float32),
            pltpu.VMEM_SHARED((NPAD, H), jnp.float32),
            pltpu.SemaphoreType.DMA,
            pltpu.SemaphoreType.DMA,
            pltpu.SemaphoreType.DMA,
            pltpu.SemaphoreType.DMA,
        ],
        compiler_params=pltpu.CompilerParams(
            needs_layout_passes=False, use_tc_tiling_on_sc=False),
    )
    def edge_kernel(pk_h, att_h, g_h, acc_h,
                    pki, attb, rowi, coli, wbuf, gb0, gb1, acc_sh,
                    gs0, gs1, as0, as1):
        cid = lax.axis_index("c")
        tid = lax.axis_index("s")
        base = tid * RPT
        kw = jnp.where(cid == 0, KA, KB)
        koff = jnp.where(cid == 0, 0, KA)

        pltpu.sync_copy(pk_h.at[tid, pl.ds(koff, KMAX)], pki)

        asems = (as0, as1)

        def att_fetch(k, bb):
            pltpu.async_copy(
                att_h.at[tid, pl.ds(koff + k, 1)],
                attb.at[pl.ds(bb, 1)], asems[bb])

        def att_wait(k, bb):
            pltpu.make_async_copy(
                att_h.at[tid, pl.ds(koff + k, 1)],
                attb.at[pl.ds(bb, 1)], asems[bb]).wait()

        def unpack(k, bb):
            for j in range(C2 // L):
                p = pki[k, pl.ds(j * L, L)]
                r = lax.bitwise_and(p, 16383)
                c = lax.shift_right_logical(p, 14)
                rowi[bb, pl.ds(j * L, L)] = r
                coli[bb, pl.ds(j * L, L)] = c

        def z_body(i, _):
            for j in range(H // L):
                gb0[i, pl.ds(j * L, L)] = jnp.zeros((L,), jnp.float32)
            return 0
        lax.fori_loop(0, C2, z_body, 0)

        def zc_body(mb, _):
            off = pl.multiple_of(base + mb * C2, C2)
            pltpu.sync_copy(gb0, acc_sh.at[pl.ds(off, C2)])
            return 0
        lax.fori_loop(0, RPT // C2, zc_body, 0)
        plsc.subcore_barrier()

        gbs = (gb0, gb1)
        gsems = (gs0, gs1)

        unpack(0, 0)
        pltpu.async_copy(g_h.at[rowi.at[0]], gb0, gs0)
        att_fetch(0, 0)
        unpack(1, 1)
        pltpu.async_copy(g_h.at[rowi.at[1]], gb1, gs1)
        att_fetch(1, 1)

        def do_chunk(k, bb):
            gbuf, gsem = gbs[bb], gsems[bb]
            pltpu.make_async_copy(g_h.at[rowi.at[bb]], gbuf, gsem).wait()
            att_wait(k, bb)

            for j in range(C2 // L):
                r = rowi[bb, pl.ds(j * L, L)]
                c = coli[bb, pl.ds(j * L, L)]
                a = attb[bb, pl.ds(j * L, L)]
                wbuf[bb, pl.ds(j * L, L)] = jnp.where(r != c, a, 0.0)

            def e_body(e, _):
                ws = plsc.load_gather(
                    wbuf, [jnp.full((L,), bb, jnp.int32),
                           jnp.full((L,), e, jnp.int32)])
                for j in range(H // L):
                    gbuf[e, pl.ds(j * L, L)] = gbuf[e, pl.ds(j * L, L)] * ws
                return 0
            lax.fori_loop(0, C2, e_body, 0)

            pltpu.sync_copy(gbuf, acc_sh.at[coli.at[bb]], add=True)

            @pl.when(k + 2 < kw)
            def _():
                unpack(k + 2, bb)
                pltpu.async_copy(g_h.at[rowi.at[bb]], gbuf, gsem)
                att_fetch(k + 2, bb)

        def loop2(k2, _):
            for bb in range(2):
                do_chunk(k2 * 2 + bb, bb)
            return 0
        lax.fori_loop(0, kw // 2, loop2, 0)

        plsc.subcore_barrier()

        def rb_body(nb, _):
            off = pl.multiple_of(base + nb * 128, 128)
            pltpu.sync_copy(acc_sh.at[pl.ds(off, 128)],
                            acc_h.at[cid, pl.ds(off, 128)])
            return 0
        lax.fori_loop(0, RPT // 128, rb_body, 0)

    return edge_kernel


def _stats_body(x_ref, na_ref, o_ref):
    i = pl.program_id(0)
    xa = na_ref[...] * x_ref[...]

    @pl.when(i == 0)
    def _():
        o_ref[...] = jnp.zeros_like(o_ref)
    o_ref[0:1, :] += jnp.sum(xa, axis=0, keepdims=True)
    o_ref[1:2, :] += jnp.sum(xa * xa, axis=0, keepdims=True)


def _deg_terms(d0, d1, l0, l1):
    lwp = jnp.maximum(l0, l1)
    lw = jnp.where(lwp >= 0.0, lwp, 1.0)
    deg = d0 + d1 + lw
    dinv = jnp.where(deg > 0.0, lax.rsqrt(jnp.maximum(deg, EPS * EPS)), 0.0)
    return lw, dinv


def _make_transform_body(N):
    def body(x_ref, na_ref, st_ref, w_ref, bnw_ref, bnb_ref,
             d0_ref, d1_ref, l0_ref, l1_ref, g_ref):
        m = st_ref[0:1, :] * (1.0 / N)
        msq = st_ref[1:2, :] * (1.0 / N)
        var = msq - m * m
        s = bnw_ref[...] * lax.rsqrt(var + EPS)
        t = bnb_ref[...] - m * s
        xb = (na_ref[...] * x_ref[...]) * s + t
        h = jnp.dot(xb, w_ref[...], preferred_element_type=jnp.float32)
        _, dinv = _deg_terms(d0_ref[...], d1_ref[...], l0_ref[...], l1_ref[...])
        g_ref[...] = dinv * h
    return body


def _pool_body(acc_ref, g_ref,
               d0_ref, d1_ref, l0_ref, l1_ref, b_ref, bat_ref, o_ref):
    i = pl.program_id(0)
    lw, dinv = _deg_terms(d0_ref[...], d1_ref[...], l0_ref[...], l1_ref[...])
    acc = acc_ref[0] + acc_ref[1]
    out = dinv * (acc + lw * g_ref[...]) + b_ref[...]
    out = jnp.maximum(out, 0.0)
    oh = (bat_ref[...] == lax.broadcasted_iota(jnp.int32, (1, G), 1))
    p = lax.dot_general(oh.astype(jnp.float32), out,
                        (((0,), (0,)), ((), ())),
                        preferred_element_type=jnp.float32)

    @pl.when(i == 0)
    def _():
        o_ref[...] = jnp.zeros_like(o_ref)
    o_ref[...] += p


def kernel(x, edge_index, batch, edge_att, node_att, W, b, bn_weight, bn_bias):
    N, H = x.shape
    E = edge_att.shape[0]
    NPAD = -(-N // (NS * 128)) * (NS * 128)

    row = edge_index[0]
    col = edge_index[1]

    KTOT = -(-E // (NS * C2))
    KTOT += KTOT % 2
    KA = (496 * KTOT) // 1000
    KA += KA % 2
    KB = KTOT - KA
    EP2 = NS * KTOT * C2
    pk_p = jnp.concatenate(
        [row | (col << 14), jnp.zeros((EP2 - E,), jnp.int32)]
    ).reshape(NS, KTOT, C2)
    att_p = jnp.concatenate(
        [edge_att, jnp.full((EP2 - E,), -1.0, jnp.float32)]
    ).reshape(NS, KTOT, C2)

    degp, lwp = _make_deg_kernel(N, NPAD, KTOT)(pk_p, att_p)

    bN = N // 5
    stats = pl.pallas_call(
        _stats_body,
        grid=(5,),
        in_specs=[pl.BlockSpec((bN, H), lambda i: (i, 0)),
                  pl.BlockSpec((bN, 1), lambda i: (i, 0))],
        out_specs=pl.BlockSpec((8, H), lambda i: (0, 0)),
        out_shape=jax.ShapeDtypeStruct((8, H), jnp.float32),
    )(x, node_att)

    d0 = degp[0, :N].reshape(N, 1)
    d1 = degp[1, :N].reshape(N, 1)
    l0 = lwp[0, :N].reshape(N, 1)
    l1 = lwp[1, :N].reshape(N, 1)

    nspec = pl.BlockSpec((bN, 1), lambda i: (i, 0))
    full2 = lambda shape: pl.BlockSpec(shape, lambda i: (0, 0))
    g = pl.pallas_call(
        _make_transform_body(N),
        grid=(5,),
        in_specs=[pl.BlockSpec((bN, H), lambda i: (i, 0)),
                  nspec,
                  full2((8, H)),
                  full2((H, H)),
                  full2((1, H)),
                  full2((1, H)),
                  nspec, nspec, nspec, nspec],
        out_specs=pl.BlockSpec((bN, H), lambda i: (i, 0)),
        out_shape=jax.ShapeDtypeStruct((N, H), jnp.float32),
    )(x, node_att, stats, W, bn_weight.reshape(1, H), bn_bias.reshape(1, H),
      d0, d1, l0, l1)

    acc = _make_edge_kernel(N, NPAD, KA, KB, H)(pk_p, att_p, g)

    aspec = pl.BlockSpec((NC, bN, H), lambda i: (0, i, 0))
    pooled = pl.pallas_call(
        _pool_body,
        grid=(5,),
        in_specs=[aspec,
                  pl.BlockSpec((bN, H), lambda i: (i, 0)),
                  nspec, nspec, nspec, nspec,
                  full2((1, H)),
                  pl.BlockSpec((bN, 1), lambda i: (i, 0))],
        out_specs=pl.BlockSpec((G, H), lambda i: (0, 0)),
        out_shape=jax.ShapeDtypeStruct((G, H), jnp.float32),
    )(acc, g, d0, d1, l0, l1,
      b.reshape(1, H), batch.reshape(N, 1))
    return pooled
